# Initial kernel scaffold; baseline (speedup 1.0000x reference)
import functools

import jax
import jax.numpy as jnp
from jax import lax
from jax.experimental import pallas as pl
from jax.experimental.pallas import tpu as pltpu
from jax.experimental.pallas import tpu_sc as plsc

B = 1024
M = 50
T = 6
D = 128
V = 100000

NC = 2
NS = 16
NW = NC * NS
ROWS = B * M
RPW = ROWS // NW
R = 80
NCH = RPW // R


def _sc_gather_sums(c0, c1, c2, idx4):
    mesh = plsc.VectorSubcoreMesh(
        core_axis_name="c", subcore_axis_name="s",
        num_cores=NC, num_subcores=NS)

    @functools.partial(
        pl.kernel,
        out_type=[jax.ShapeDtypeStruct((ROWS, D), jnp.float32)] * 3,
        mesh=mesh,
        scratch_types=[
            pltpu.VMEM((NCH, T, R), jnp.int32),
            pltpu.VMEM((R, D), jnp.float32),
            pltpu.VMEM((R, D), jnp.float32),
            pltpu.VMEM((R, D), jnp.float32),
            pltpu.SemaphoreType.DMA,
        ],
    )
    def k(c0h, c1h, c2h, idx_hbm, s0h, s1h, s2h, idx_v, d0, d1, d2, sem):
        wid = lax.axis_index("s") * NC + lax.axis_index("c")
        base = wid * RPW
        pltpu.sync_copy(idx_hbm.at[wid], idx_v)
        tabs = (c0h, c1h, c2h)
        outs = (s0h, s1h, s2h)
        dsts = (d0, d1, d2)

        def chunk(c, carry):
            cps = [pltpu.async_copy(tabs[h].at[idx_v.at[c, 0]], dsts[h], sem)
                   for h in range(3)]
            for cp in cps:
                cp.wait()
            cps = [pltpu.async_copy(tabs[h].at[idx_v.at[c, t]], dsts[h], sem,
                                    add=True)
                   for h in range(3) for t in range(1, T)]
            for cp in cps:
                cp.wait()
            for h in range(3):
                pltpu.sync_copy(dsts[h], outs[h].at[pl.ds(base + c * R, R)])
            return carry

        lax.fori_loop(0, NCH, chunk, 0)

    return k(c0, c1, c2, idx4)


BB = 64


def _tc_body(kb_ref, cv_ref, gp_ref, qv_ref, s0_ref, s1_ref, s2_ref, dh_ref,
             ps_ref, lg_ref):
    f32 = jnp.float32
    kb3 = kb_ref[...]
    cv3 = cv_ref[...]
    m_i = lax.broadcasted_iota(jnp.int32, (BB, M, M), 1)
    j_i = lax.broadcasted_iota(jnp.int32, (BB, M, M), 2)
    w = ((m_i - j_i == kb3) & (j_i < cv3)).astype(f32)
    dh = dh_ref[...]
    lm = lax.dot_general(w, dh, (((2,), (1,)), ((0,), (0,))),
                         preferred_element_type=f32)
    gp = gp_ref[...]
    e0 = (s0_ref[...] + lm) * gp
    e1 = (s1_ref[...] + lm) * gp
    e2 = (s2_ref[...] + lm) * gp
    qv = qv_ref[...]

    def soft(lg):
        mx = jnp.max(lg, axis=1, keepdims=True)
        ex = jnp.exp(lg - mx)
        return ex / jnp.sum(ex, axis=1, keepdims=True)

    l0 = jnp.sum(e0 * qv, axis=2, keepdims=True)
    p0 = soft(l0)
    q1 = qv + jnp.sum(e1 * p0, axis=1, keepdims=True)
    l1 = jnp.sum(e1 * q1, axis=2, keepdims=True)
    p1 = soft(l1)
    q2 = q1 + jnp.sum(e2 * p1, axis=1, keepdims=True)
    l2 = jnp.sum(e2 * q2, axis=2, keepdims=True)
    ps_ref[...] = soft(l2)
    lg_ref[...] = l2


def _tc_forward(kb, cv, gp3, qv3, s0, s1, s2, dh):
    grid = (B // BB,)
    bs3 = pl.BlockSpec((BB, M, D), lambda i: (i, 0, 0))
    out_shape = [jax.ShapeDtypeStruct((B, M, 1), jnp.float32)] * 2
    return pl.pallas_call(
        _tc_body,
        grid=grid,
        in_specs=[
            pl.BlockSpec((BB, 1, 1), lambda i: (i, 0, 0)),
            pl.BlockSpec((BB, 1, 1), lambda i: (i, 0, 0)),
            pl.BlockSpec((BB, M, 1), lambda i: (i, 0, 0)),
            pl.BlockSpec((BB, 1, D), lambda i: (i, 0, 0)),
            bs3, bs3, bs3, bs3,
        ],
        out_specs=[pl.BlockSpec((BB, M, 1), lambda i: (i, 0, 0))] * 2,
        out_shape=out_shape,
    )(kb, cv, gp3, qv3, s0, s1, s2, dh)


def kernel(story, kb_len, conv_len, hidden, dh_outputs, query_vector,
           global_pointer, C0, C1, C2, C3):
    del hidden, C3
    idx4 = (story.astype(jnp.int32)
            .reshape(ROWS, T)
            .reshape(NW, NCH, R, T)
            .transpose(0, 1, 3, 2))
    s0, s1, s2 = _sc_gather_sums(C0, C1, C2, idx4)
    s0 = s0.reshape(B, M, D)
    s1 = s1.reshape(B, M, D)
    s2 = s2.reshape(B, M, D)
    kb = kb_len.astype(jnp.int32).reshape(B, 1, 1)
    cv = conv_len.astype(jnp.int32).reshape(B, 1, 1)
    gp3 = global_pointer.reshape(B, M, 1)
    qv3 = query_vector.reshape(B, 1, D)
    ps, lg = _tc_forward(kb, cv, gp3, qv3, s0, s1, s2, dh_outputs)
    return ps.reshape(B, M), lg.reshape(B, M)


# speedup vs baseline: 9.8745x; 9.8745x over previous
import functools

import jax
import jax.numpy as jnp
from jax import lax
from jax.experimental import pallas as pl
from jax.experimental.pallas import tpu as pltpu
from jax.experimental.pallas import tpu_sc as plsc

B = 1024
M = 50
T = 6
D = 128
V = 100000

NC = 2
NS = 16
NW = NC * NS
ROWS = B * M
RPW = ROWS // NW
R = 80
NCH = RPW // R


def _sc_gather_sums(c0, c1, c2, idx4):
    mesh = plsc.VectorSubcoreMesh(
        core_axis_name="c", subcore_axis_name="s",
        num_cores=NC, num_subcores=NS)

    @functools.partial(
        pl.kernel,
        out_type=[jax.ShapeDtypeStruct((ROWS, D), jnp.float32)] * 3,
        mesh=mesh,
        scratch_types=[
            pltpu.VMEM((NCH, T, R), jnp.int32),
            pltpu.VMEM((R, D), jnp.float32),
            pltpu.VMEM((R, D), jnp.float32),
            pltpu.VMEM((R, D), jnp.float32),
            pltpu.SemaphoreType.DMA,
        ],
    )
    def k(c0h, c1h, c2h, idx_hbm, s0h, s1h, s2h, idx_v, d0, d1, d2, sem):
        wid = lax.axis_index("s") * NC + lax.axis_index("c")
        base = wid * RPW
        pltpu.sync_copy(idx_hbm.at[wid], idx_v)
        tabs = (c0h, c1h, c2h)
        outs = (s0h, s1h, s2h)
        dsts = (d0, d1, d2)

        def chunk(c, carry):
            cps = [pltpu.async_copy(tabs[h].at[idx_v.at[c, 0]], dsts[h], sem)
                   for h in range(3)]
            for cp in cps:
                cp.wait()
            cps = [pltpu.async_copy(tabs[h].at[idx_v.at[c, t]], dsts[h], sem,
                                    add=True)
                   for h in range(3) for t in range(1, T)]
            for cp in cps:
                cp.wait()
            for h in range(3):
                pltpu.sync_copy(dsts[h], outs[h].at[pl.ds(base + c * R, R)])
            return carry

        lax.fori_loop(0, NCH, chunk, 0)

    return k(c0, c1, c2, idx4)


BB = 64


def _tc_body(kb_ref, cv_ref, gp_ref, qv_ref, s0_ref, s1_ref, s2_ref, dh_ref,
             ps_ref, lg_ref):
    f32 = jnp.float32
    kb3 = kb_ref[...]
    cv3 = cv_ref[...]
    m_i = lax.broadcasted_iota(jnp.int32, (BB, M, M), 1)
    j_i = lax.broadcasted_iota(jnp.int32, (BB, M, M), 2)
    w = ((m_i - j_i == kb3) & (j_i < cv3)).astype(f32)
    dh = dh_ref[...]
    lm = lax.dot_general(w, dh, (((2,), (1,)), ((0,), (0,))),
                         precision=lax.Precision.HIGHEST,
                         preferred_element_type=f32)
    gp = gp_ref[...]
    e0 = (s0_ref[...] + lm) * gp
    e1 = (s1_ref[...] + lm) * gp
    e2 = (s2_ref[...] + lm) * gp
    qv = qv_ref[...]

    def soft(lg):
        mx = jnp.max(lg, axis=1, keepdims=True)
        ex = jnp.exp(lg - mx)
        return ex / jnp.sum(ex, axis=1, keepdims=True)

    l0 = jnp.sum(e0 * qv, axis=2, keepdims=True)
    p0 = soft(l0)
    q1 = qv + jnp.sum(e1 * p0, axis=1, keepdims=True)
    l1 = jnp.sum(e1 * q1, axis=2, keepdims=True)
    p1 = soft(l1)
    q2 = q1 + jnp.sum(e2 * p1, axis=1, keepdims=True)
    l2 = jnp.sum(e2 * q2, axis=2, keepdims=True)
    ps_ref[...] = soft(l2)
    lg_ref[...] = l2


def _tc_forward(kb, cv, gp3, qv3, s0, s1, s2, dh):
    grid = (B // BB,)
    bs3 = pl.BlockSpec((BB, M, D), lambda i: (i, 0, 0))
    out_shape = [jax.ShapeDtypeStruct((B, M, 1), jnp.float32)] * 2
    return pl.pallas_call(
        _tc_body,
        grid=grid,
        in_specs=[
            pl.BlockSpec((BB, 1, 1), lambda i: (i, 0, 0)),
            pl.BlockSpec((BB, 1, 1), lambda i: (i, 0, 0)),
            pl.BlockSpec((BB, M, 1), lambda i: (i, 0, 0)),
            pl.BlockSpec((BB, 1, D), lambda i: (i, 0, 0)),
            bs3, bs3, bs3, bs3,
        ],
        out_specs=[pl.BlockSpec((BB, M, 1), lambda i: (i, 0, 0))] * 2,
        out_shape=out_shape,
    )(kb, cv, gp3, qv3, s0, s1, s2, dh)


def kernel(story, kb_len, conv_len, hidden, dh_outputs, query_vector,
           global_pointer, C0, C1, C2, C3):
    del hidden, C3
    idx4 = (story.astype(jnp.int32)
            .reshape(ROWS, T)
            .reshape(NW, NCH, R, T)
            .transpose(0, 1, 3, 2))
    s0, s1, s2 = _sc_gather_sums(C0, C1, C2, idx4)
    s0 = s0.reshape(B, M, D)
    s1 = s1.reshape(B, M, D)
    s2 = s2.reshape(B, M, D)
    kb = kb_len.astype(jnp.int32).reshape(B, 1, 1)
    cv = conv_len.astype(jnp.int32).reshape(B, 1, 1)
    gp3 = global_pointer.reshape(B, M, 1)
    qv3 = query_vector.reshape(B, 1, D)
    ps, lg = _tc_forward(kb, cv, gp3, qv3, s0, s1, s2, dh_outputs)
    return ps.reshape(B, M), lg.reshape(B, M)


# speedup vs baseline: 14.3149x; 1.4497x over previous
import functools

import jax
import jax.numpy as jnp
from jax import lax
from jax.experimental import pallas as pl
from jax.experimental.pallas import tpu as pltpu
from jax.experimental.pallas import tpu_sc as plsc

B = 1024
M = 50
MP = 64
T = 6
D = 128
V = 100000

NC = 2
NS = 16
NW = NC * NS
BPW = B // NW
CB = 2
RCH = CB * M
NCH = BPW // CB
WR = 56
RBUF = M + WR


def _sc_gather_sums(c0, c1, c2, idx4):
    mesh = plsc.VectorSubcoreMesh(
        core_axis_name="c", subcore_axis_name="s",
        num_cores=NC, num_subcores=NS)

    @functools.partial(
        pl.kernel,
        out_type=[jax.ShapeDtypeStruct((B * MP, D), jnp.float32)] * 3,
        mesh=mesh,
        scratch_types=[
            pltpu.VMEM((NCH, T, RCH), jnp.int32),
            pltpu.VMEM((RBUF, D), jnp.float32),
            pltpu.VMEM((RBUF, D), jnp.float32),
            pltpu.VMEM((RBUF, D), jnp.float32),
            pltpu.VMEM((RBUF, D), jnp.float32),
            pltpu.VMEM((RBUF, D), jnp.float32),
            pltpu.VMEM((RBUF, D), jnp.float32),
            pltpu.SemaphoreType.DMA,
            pltpu.SemaphoreType.DMA,
            pltpu.SemaphoreType.DMA,
            pltpu.SemaphoreType.DMA,
            pltpu.SemaphoreType.DMA,
            pltpu.SemaphoreType.DMA,
        ],
    )
    def k(c0h, c1h, c2h, idx_hbm, s0h, s1h, s2h, idx_v,
          a0, a1, a2, b0, b1, b2,
          sw1a, sw1b, sw2a, sw2b, soa, sob):
        wid = lax.axis_index("s") * NC + lax.axis_index("c")
        pltpu.sync_copy(idx_hbm.at[wid], idx_v)
        tabs = (c0h, c1h, c2h)
        outs = (s0h, s1h, s2h)
        bufa = (a0, a1, a2)
        bufb = (b0, b1, b2)

        def w1(c, bufs, sem):
            for h in range(3):
                pltpu.async_copy(tabs[h].at[idx_v.at[c, 0]],
                                 bufs[h].at[pl.ds(0, RCH)], sem)

        def w1_wait(c, bufs, sem):
            for h in range(3):
                pltpu.make_async_copy(tabs[h].at[idx_v.at[c, 0]],
                                      bufs[h].at[pl.ds(0, RCH)], sem).wait()

        def w2(c, bufs, sem):
            for h in range(3):
                for t in range(1, T):
                    pltpu.async_copy(tabs[h].at[idx_v.at[c, t]],
                                     bufs[h].at[pl.ds(0, RCH)], sem, add=True)

        def w2_wait(c, bufs, sem):
            for h in range(3):
                for t in range(1, T):
                    pltpu.make_async_copy(tabs[h].at[idx_v.at[c, t]],
                                          bufs[h].at[pl.ds(0, RCH)],
                                          sem).wait()

        def outw(c, bufs, sem):
            b0r = (wid * BPW + c * CB) * MP
            for h in range(3):
                pltpu.async_copy(bufs[h].at[pl.ds(0, WR)],
                                 outs[h].at[pl.ds(b0r, WR)], sem)
                pltpu.async_copy(bufs[h].at[pl.ds(M, WR)],
                                 outs[h].at[pl.ds(b0r + MP, WR)], sem)

        def outw_wait(c, bufs, sem):
            b0r = (wid * BPW + c * CB) * MP
            for h in range(3):
                pltpu.make_async_copy(bufs[h].at[pl.ds(0, WR)],
                                      outs[h].at[pl.ds(b0r, WR)], sem).wait()
                pltpu.make_async_copy(bufs[h].at[pl.ds(M, WR)],
                                      outs[h].at[pl.ds(b0r + MP, WR)],
                                      sem).wait()

        w1(0, bufa, sw1a)
        w1_wait(0, bufa, sw1a)
        w2(0, bufa, sw2a)
        w1(1, bufb, sw1b)
        w2_wait(0, bufa, sw2a)
        outw(0, bufa, soa)
        w1_wait(1, bufb, sw1b)
        w2(1, bufb, sw2b)
        outw_wait(0, bufa, soa)
        w1(2, bufa, sw1a)
        w2_wait(1, bufb, sw2b)
        outw(1, bufb, sob)

        def pair(i, carry):
            ca = 2 * i
            w1_wait(ca, bufa, sw1a)
            w2(ca, bufa, sw2a)
            outw_wait(ca - 1, bufb, sob)
            w1(ca + 1, bufb, sw1b)
            w2_wait(ca, bufa, sw2a)
            outw(ca, bufa, soa)
            w1_wait(ca + 1, bufb, sw1b)
            w2(ca + 1, bufb, sw2b)
            outw_wait(ca, bufa, soa)
            w1(ca + 2, bufa, sw1a)
            w2_wait(ca + 1, bufb, sw2b)
            outw(ca + 1, bufb, sob)
            return carry

        lax.fori_loop(1, NCH // 2 - 1, pair, 0)

        ca = NCH - 2
        w1_wait(ca, bufa, sw1a)
        w2(ca, bufa, sw2a)
        outw_wait(ca - 1, bufb, sob)
        w1(ca + 1, bufb, sw1b)
        w2_wait(ca, bufa, sw2a)
        outw(ca, bufa, soa)
        w1_wait(ca + 1, bufb, sw1b)
        w2(ca + 1, bufb, sw2b)
        outw_wait(ca, bufa, soa)
        w2_wait(ca + 1, bufb, sw2b)
        outw(ca + 1, bufb, sob)
        outw_wait(ca + 1, bufb, sob)

    return k(c0, c1, c2, idx4)


BB = 64
NEG = -1e30


def _tc_body(kb_ref, cv_ref, gp_ref, qv_ref, s0_ref, s1_ref, s2_ref, dh_ref,
             ps_ref, lg_ref):
    f32 = jnp.float32
    kb3 = kb_ref[...][:, :, None]
    cv3 = cv_ref[...][:, :, None]
    m_i = lax.broadcasted_iota(jnp.int32, (BB, MP, M), 1)
    j_i = lax.broadcasted_iota(jnp.int32, (BB, MP, M), 2)
    w = ((m_i - j_i == kb3) & (j_i < cv3)).astype(f32)
    dh = dh_ref[...]
    lm = lax.dot_general(w, dh, (((2,), (1,)), ((0,), (0,))),
                         precision=lax.Precision.HIGHEST,
                         preferred_element_type=f32)
    rowmask = lax.broadcasted_iota(jnp.int32, (BB, MP, D), 1) < M
    e0 = jnp.where(rowmask, s0_ref[...] + lm, 0.0)
    e1 = jnp.where(rowmask, s1_ref[...] + lm, 0.0)
    e2 = jnp.where(rowmask, s2_ref[...] + lm, 0.0)
    gp = gp_ref[...]
    qv = qv_ref[...]
    lanemask = lax.broadcasted_iota(jnp.int32, (BB, MP), 1) < M

    def logits(e, q):
        raw = jnp.sum(e * q, axis=2)
        return jnp.where(lanemask, gp * raw, NEG)

    def soft(lg):
        mx = jnp.max(lg, axis=1, keepdims=True)
        ex = jnp.exp(lg - mx)
        return ex / jnp.sum(ex, axis=1, keepdims=True)

    def attend(e, p):
        return jnp.sum(e * (p * gp)[:, :, None], axis=1, keepdims=True)

    l0 = logits(e0, qv)
    q1 = qv + attend(e1, soft(l0))
    l1 = logits(e1, q1)
    q2 = q1 + attend(e2, soft(l1))
    l2 = logits(e2, q2)
    ps_ref[...] = soft(l2)[:, :M]
    lg_ref[...] = l2[:, :M]


def _tc_forward(kb, cv, gp2, qv3, s0, s1, s2, dh):
    grid = (B // BB,)
    bs3 = pl.BlockSpec((BB, MP, D), lambda i: (i, 0, 0))
    return pl.pallas_call(
        _tc_body,
        grid=grid,
        in_specs=[
            pl.BlockSpec((BB, 1), lambda i: (i, 0)),
            pl.BlockSpec((BB, 1), lambda i: (i, 0)),
            pl.BlockSpec((BB, MP), lambda i: (i, 0)),
            pl.BlockSpec((BB, 1, D), lambda i: (i, 0, 0)),
            bs3, bs3, bs3,
            pl.BlockSpec((BB, M, D), lambda i: (i, 0, 0)),
        ],
        out_specs=[pl.BlockSpec((BB, M), lambda i: (i, 0))] * 2,
        out_shape=[jax.ShapeDtypeStruct((B, M), jnp.float32)] * 2,
    )(kb, cv, gp2, qv3, s0, s1, s2, dh)


def kernel(story, kb_len, conv_len, hidden, dh_outputs, query_vector,
           global_pointer, C0, C1, C2, C3):
    del hidden, C3
    idx4 = (story.astype(jnp.int32)
            .reshape(B * M, T)
            .reshape(NW, NCH, RCH, T)
            .transpose(0, 1, 3, 2))
    s0, s1, s2 = _sc_gather_sums(C0, C1, C2, idx4)
    s0 = s0.reshape(B, MP, D)
    s1 = s1.reshape(B, MP, D)
    s2 = s2.reshape(B, MP, D)
    kb = kb_len.astype(jnp.int32).reshape(B, 1)
    cv = conv_len.astype(jnp.int32).reshape(B, 1)
    gp2 = jnp.pad(global_pointer, ((0, 0), (0, MP - M)))
    qv3 = query_vector.reshape(B, 1, D)
    return _tc_forward(kb, cv, gp2, qv3, s0, s1, s2, dh_outputs)
